# baseline (speedup 1.0000x reference)
import functools

import jax
import jax.numpy as jnp
from jax import lax
from jax.experimental import pallas as pl
from jax.experimental.pallas import tpu as pltpu
from jax.experimental.pallas import tpu_sc as plsc

GAMMA_C = 12.0
BASE_DIM_C = 64
LANES = 16
NUM_CORES = 2
NUM_SUBCORES = 16
NUM_WORKERS = NUM_CORES * NUM_SUBCORES


def _build(batch, dim):
    b_per_w = batch // NUM_WORKERS
    groups = b_per_w // LANES

    mesh = plsc.VectorSubcoreMesh(core_axis_name="c", subcore_axis_name="s")

    @functools.partial(
        pl.kernel,
        mesh=mesh,
        compiler_params=pltpu.CompilerParams(
            needs_layout_passes=False, use_tc_tiling_on_sc=False),
        out_type=jax.ShapeDtypeStruct((batch,), jnp.float32),
        scratch_types=[
            pltpu.VMEM((b_per_w,), jnp.int32),
            pltpu.VMEM((b_per_w,), jnp.int32),
            pltpu.VMEM((b_per_w,), jnp.int32),
            pltpu.VMEM((b_per_w, dim), jnp.float32),
            pltpu.VMEM((b_per_w, dim), jnp.float32),
            pltpu.VMEM((b_per_w, dim), jnp.float32),
            pltpu.VMEM((b_per_w,), jnp.float32),
            pltpu.SemaphoreType.DMA,
            pltpu.SemaphoreType.DMA,
            pltpu.SemaphoreType.DMA,
        ],
    )
    def kg_score(h_idx_hbm, r_idx_hbm, t_idx_hbm, ent_hbm, rel_hbm, out_hbm,
                 h_idx_v, r_idx_v, t_idx_v, h_rows, r_rows, t_rows, out_v,
                 sem_h, sem_r, sem_t):
        wid = lax.axis_index("s") * NUM_CORES + lax.axis_index("c")
        base = wid * b_per_w

        pltpu.sync_copy(h_idx_hbm.at[pl.ds(base, b_per_w)], h_idx_v)
        pltpu.sync_copy(r_idx_hbm.at[pl.ds(base, b_per_w)], r_idx_v)
        pltpu.sync_copy(t_idx_hbm.at[pl.ds(base, b_per_w)], t_idx_v)

        cp_h = pltpu.async_copy(ent_hbm.at[h_idx_v], h_rows, sem_h)
        cp_r = pltpu.async_copy(rel_hbm.at[r_idx_v], r_rows, sem_r)
        cp_t = pltpu.async_copy(ent_hbm.at[t_idx_v], t_rows, sem_t)
        cp_h.wait()
        cp_r.wait()
        cp_t.wait()

        lanes = lax.iota(jnp.int32, LANES)

        def g_body(g, _):
            row = lanes + g * LANES

            def d_body(d, acc):
                col = jnp.full((LANES,), d, jnp.int32)
                hv = plsc.load_gather(h_rows, [row, col])
                rv = plsc.load_gather(r_rows, [row, col])
                tv = plsc.load_gather(t_rows, [row, col])
                return acc + jnp.abs(hv + rv - tv)

            acc = lax.fori_loop(0, dim, d_body, jnp.zeros((LANES,), jnp.float32))
            out_v[pl.ds(pl.multiple_of(g * LANES, LANES), LANES)] = GAMMA_C - acc
            return 0

        lax.fori_loop(0, groups, g_body, 0)

        pltpu.sync_copy(out_v, out_hbm.at[pl.ds(base, b_per_w)])

    return kg_score


def kernel(sample, entity_embedding, relation_embedding):
    batch = sample.shape[0]
    dim = entity_embedding.shape[1]
    s32 = sample.astype(jnp.int32)
    h_idx = s32[:, 0]
    r_idx = s32[:, 1]
    t_idx = s32[:, 2]
    score = _build(batch, dim)(h_idx, r_idx, t_idx,
                               entity_embedding, relation_embedding)
    return score[:, None]
